# baseline (speedup 1.0000x reference)
import functools
import numpy as np
import jax
import jax.numpy as jnp
from jax import lax
from jax.experimental import pallas as pl
from jax.experimental.pallas import tpu as pltpu
from jax.experimental.pallas import tpu_sc as plsc

_NN = 4
_S = 200
_D = 64
_B = 4096
_T = _S * _NN

_NC = 2
_NS = 16
_NW = _NC * _NS

_BPW = _B // _NW
_BG = _BPW // 16


def _pe_flat_np():
    i_num = np.arange(0.0, _S, dtype=np.float32).reshape(-1, 1)
    j_denom = np.power(
        10000.0, np.arange(0.0, _D, 2.0, dtype=np.float32) / _D
    )
    pe = np.zeros((_S, _D), dtype=np.float32)
    pe[:, 0::2] = np.sin(i_num / j_denom)
    pe[:, 1::2] = np.cos(i_num / j_denom)
    return pe.reshape(-1)


def _sc_body(x_hbm, pe_hbm, emb_hbm, out_hbm,
             x_all, tt_v, pe_v, emb_v, obuf0, obuf1, sem_o0, sem_o1):
    cid = lax.axis_index("c")
    sid = lax.axis_index("s")
    wid = sid * _NC + cid
    b0 = wid * _BPW

    lane = lax.iota(jnp.int32, 16)

    pltpu.sync_copy(x_hbm.at[pl.ds(b0 * _S, _BPW * _S)], x_all)
    pltpu.sync_copy(pe_hbm, pe_v)
    pltpu.sync_copy(emb_hbm, emb_v)

    def build_d(d, carry):
        for tc in range(_T // 16):
            t16 = tc * 16 + lane
            s16 = lax.shift_right_logical(t16, 2)
            v16 = lax.bitwise_and(t16, 3)
            pe16 = plsc.load_gather(pe_v, [s16 * _D + d])
            e16 = plsc.load_gather(emb_v, [v16 * _D + d])
            tt_v[pl.ds(d * _T + tc * 16, 16)] = pe16 + e16
        return carry

    lax.fori_loop(0, _D, build_d, 0)

    lane200 = lane * _S

    def do_s(s, i, obuf, sem_o):
        @pl.when(i > 0)
        def _():
            pltpu.make_async_copy(
                obuf, out_hbm.at[pl.ds(0, 1), :, pl.ds(b0, _BPW)], sem_o
            ).wait()
        for bg in range(_BG):
            x16 = plsc.load_gather(x_all, [lane200 + (bg * 16 * _S + s)])
            idx = s * _NN + x16
            for d in range(_D):
                g16 = plsc.load_gather(tt_v, [idx])
                obuf[0, d, pl.ds(bg * 16, 16)] = g16
                idx = idx + _T
        pltpu.async_copy(
            obuf, out_hbm.at[pl.ds(s, 1), :, pl.ds(b0, _BPW)], sem_o
        )

    def pair_body(i, carry):
        do_s(2 * i, i, obuf0, sem_o0)
        do_s(2 * i + 1, i, obuf1, sem_o1)
        return carry

    lax.fori_loop(0, _S // 2, pair_body, 0)
    for obuf, sem_o in ((obuf0, sem_o0), (obuf1, sem_o1)):
        pltpu.make_async_copy(
            obuf, out_hbm.at[pl.ds(0, 1), :, pl.ds(b0, _BPW)], sem_o
        ).wait()


@jax.jit
def kernel(X, emb):
    X = X.astype(jnp.int32).reshape(-1)
    pe_flat = jnp.asarray(_pe_flat_np())
    emb_flat = emb.reshape(-1)
    mesh = plsc.VectorSubcoreMesh(core_axis_name="c", subcore_axis_name="s")
    out = pl.kernel(
        _sc_body,
        mesh=mesh,
        compiler_params=pltpu.CompilerParams(
            use_tc_tiling_on_sc=True, needs_layout_passes=False
        ),
        out_type=jax.ShapeDtypeStruct((_S, _D, _B), jnp.float32),
        scratch_types=[
            pltpu.VMEM((_BPW * _S,), jnp.int32),
            pltpu.VMEM((_D * _T,), jnp.float32),
            pltpu.VMEM((_S * _D,), jnp.float32),
            pltpu.VMEM((_NN * _D,), jnp.float32),
            pltpu.VMEM((1, _D, _BPW), jnp.float32),
            pltpu.VMEM((1, _D, _BPW), jnp.float32),
            pltpu.SemaphoreType.DMA,
            pltpu.SemaphoreType.DMA,
        ],
    )(X, pe_flat, emb_flat)
    return jnp.transpose(out, (2, 0, 1))


# speedup vs baseline: 3.5201x; 3.5201x over previous
import functools
import numpy as np
import jax
import jax.numpy as jnp
from jax import lax
from jax.experimental import pallas as pl
from jax.experimental.pallas import tpu as pltpu
from jax.experimental.pallas import tpu_sc as plsc

_NN = 4
_S = 200
_D = 64
_B = 4096
_T = _S * _NN

_NC = 2
_NS = 16
_NW = _NC * _NS

_BPW = _B // _NW
_BG = _BPW // 16


def _pe_flat_np():
    i_num = np.arange(0.0, _S, dtype=np.float32).reshape(-1, 1)
    j_denom = np.power(
        10000.0, np.arange(0.0, _D, 2.0, dtype=np.float32) / _D
    )
    pe = np.zeros((_S, _D), dtype=np.float32)
    pe[:, 0::2] = np.sin(i_num / j_denom)
    pe[:, 1::2] = np.cos(i_num / j_denom)
    return pe.reshape(-1)


def _sc_body(x_hbm, pe_hbm, emb_hbm, out_hbm,
             x_all, tt_v, pe_v, emb_v, obuf0, obuf1, sem_o0, sem_o1):
    cid = lax.axis_index("c")
    sid = lax.axis_index("s")
    wid = sid * _NC + cid
    b0 = wid * _BPW

    lane = lax.iota(jnp.int32, 16)

    pltpu.sync_copy(x_hbm.at[pl.ds(b0 * _S, _BPW * _S)], x_all)
    pltpu.sync_copy(pe_hbm, pe_v)
    pltpu.sync_copy(emb_hbm, emb_v)

    @plsc.parallel_loop(0, _D, unroll=4)
    def build_d(d):
        for tc in range(_T // 16):
            t16 = tc * 16 + lane
            s16 = lax.shift_right_logical(t16, 2)
            v16 = lax.bitwise_and(t16, 3)
            pe16 = plsc.load_gather(pe_v, [s16 * _D + d])
            e16 = plsc.load_gather(emb_v, [v16 * _D + d])
            tt_v[pl.ds(d * _T + tc * 16, 16)] = pe16 + e16

    lane200 = lane * _S

    def do_s(s, i, obuf, sem_o):
        @pl.when(i > 0)
        def _():
            pltpu.make_async_copy(
                obuf, out_hbm.at[pl.ds(0, 1), :, pl.ds(b0, _BPW)], sem_o
            ).wait()
        idxs = tuple(
            s * _NN
            + plsc.load_gather(x_all, [lane200 + (bg * 16 * _S + s)])
            for bg in range(_BG)
        )

        @plsc.parallel_loop(0, _D, unroll=4, carry=idxs)
        def gather_d(d, idxs):
            for bg in range(_BG):
                g16 = plsc.load_gather(tt_v, [idxs[bg]])
                obuf[0, d, pl.ds(bg * 16, 16)] = g16
            return tuple(idx + _T for idx in idxs)
        pltpu.async_copy(
            obuf, out_hbm.at[pl.ds(s, 1), :, pl.ds(b0, _BPW)], sem_o
        )

    def pair_body(i, carry):
        do_s(2 * i, i, obuf0, sem_o0)
        do_s(2 * i + 1, i, obuf1, sem_o1)
        return carry

    lax.fori_loop(0, _S // 2, pair_body, 0)
    for obuf, sem_o in ((obuf0, sem_o0), (obuf1, sem_o1)):
        pltpu.make_async_copy(
            obuf, out_hbm.at[pl.ds(0, 1), :, pl.ds(b0, _BPW)], sem_o
        ).wait()


@jax.jit
def kernel(X, emb):
    X = X.astype(jnp.int32).reshape(-1)
    pe_flat = jnp.asarray(_pe_flat_np())
    emb_flat = emb.reshape(-1)
    mesh = plsc.VectorSubcoreMesh(core_axis_name="c", subcore_axis_name="s")
    out = pl.kernel(
        _sc_body,
        mesh=mesh,
        compiler_params=pltpu.CompilerParams(
            use_tc_tiling_on_sc=True, needs_layout_passes=False
        ),
        out_type=jax.ShapeDtypeStruct((_S, _D, _B), jnp.float32),
        scratch_types=[
            pltpu.VMEM((_BPW * _S,), jnp.int32),
            pltpu.VMEM((_D * _T,), jnp.float32),
            pltpu.VMEM((_S * _D,), jnp.float32),
            pltpu.VMEM((_NN * _D,), jnp.float32),
            pltpu.VMEM((1, _D, _BPW), jnp.float32),
            pltpu.VMEM((1, _D, _BPW), jnp.float32),
            pltpu.SemaphoreType.DMA,
            pltpu.SemaphoreType.DMA,
        ],
    )(X, pe_flat, emb_flat)
    return jnp.transpose(out, (2, 0, 1))
